# baseline (speedup 1.0000x reference)
import functools

import jax
import jax.numpy as jnp
from jax import lax
from jax.experimental import pallas as pl
from jax.experimental.pallas import tpu as pltpu
from jax.experimental.pallas import tpu_sc as plsc

N = 200000
M = N // 400
L = 16
NC = 2
NS = 16
NW = NC * NS
P = 6272
NV = P // L
UNROLL = 4
NPAD = NW * P
RW = L
OUT_PAD = 512

F32 = jnp.float32
I32 = jnp.int32
POS_INF = float("inf")
NEG_INF = float("-inf")
SENT_IDX = 1_000_000_000


def _splat_f(x):
    return jnp.full((L,), x, dtype=F32)


def _splat_i(x):
    return jnp.full((L,), x, dtype=I32)


_mesh = plsc.VectorSubcoreMesh(core_axis_name="c", subcore_axis_name="s")


@functools.partial(
    pl.kernel,
    out_type=jax.ShapeDtypeStruct((OUT_PAD,), I32),
    mesh=_mesh,
    scratch_types=[
        pltpu.VMEM((P,), F32),
        pltpu.VMEM((P,), F32),
        pltpu.VMEM((P,), F32),
        pltpu.VMEM((P,), F32),
        pltpu.VMEM((L,), F32),
        pltpu.VMEM((NW, RW), F32),
        pltpu.VMEM((OUT_PAD,), I32),
        pltpu.HBM((NW, RW), F32),
        pltpu.HBM((NW, RW), F32),
    ],
    compiler_params=pltpu.CompilerParams(needs_layout_passes=False),
)
def _fps_sc(pt_hbm, out_hbm, px, py, pz, dists, stage, rows, outb, ex0, ex1):
    w = lax.axis_index("c") * NS + lax.axis_index("s")
    base = w * P
    iota = lax.iota(I32, L)
    neg = _splat_f(NEG_INF)

    pltpu.sync_copy(pt_hbm.at[pl.ds(0 * NPAD + base, P)], px)
    pltpu.sync_copy(pt_hbm.at[pl.ds(1 * NPAD + base, P)], py)
    pltpu.sync_copy(pt_hbm.at[pl.ds(2 * NPAD + base, P)], pz)

    def _init(j, c):
        gi = _splat_i(base + j * L) + iota
        dists[pl.ds(j * L, L)] = jnp.where(gi < _splat_i(N), _splat_f(POS_INF),
                                           neg)
        return c

    lax.fori_loop(0, NV, _init, 0)

    def publish(buf, tag, val, idx, cx, cy, cz):
        vec = _splat_f(cx)
        vec = jnp.where(iota == _splat_i(1), _splat_f(cy), vec)
        vec = jnp.where(iota == _splat_i(2), _splat_f(cz), vec)
        vec = jnp.where(iota == _splat_i(3), _splat_f(val), vec)
        vec = jnp.where(iota == _splat_i(4), _splat_f(idx.astype(F32)), vec)
        vec = jnp.where(iota == _splat_i(5), _splat_f(tag.astype(F32)), vec)
        stage[...] = vec

        @pl.when(buf == 0)
        def _pub0():
            pltpu.sync_copy(stage, ex0.at[w, pl.ds(0, L)])

        @pl.when(buf == 1)
        def _pub1():
            pltpu.sync_copy(stage, ex1.at[w, pl.ds(0, L)])

    def winner(buf, tag):
        tgt = _splat_f(tag.astype(F32))

        def _cond(ok):
            return jnp.logical_not(ok)

        def _body(ok):
            @pl.when(buf == 0)
            def _rd0():
                pltpu.sync_copy(ex0, rows)

            @pl.when(buf == 1)
            def _rd1():
                pltpu.sync_copy(ex1, rows)

            t1 = plsc.load_gather(rows, [iota, _splat_i(5)])
            t2 = plsc.load_gather(rows, [iota + _splat_i(NS), _splat_i(5)])
            return jnp.all((t1 == tgt) & (t2 == tgt))

        lax.while_loop(_cond, _body, jnp.full((), False))

        io2 = iota + _splat_i(NS)
        v1 = plsc.load_gather(rows, [iota, _splat_i(3)])
        v2 = plsc.load_gather(rows, [io2, _splat_i(3)])
        i1 = plsc.load_gather(rows, [iota, _splat_i(4)]).astype(I32)
        i2 = plsc.load_gather(rows, [io2, _splat_i(4)]).astype(I32)
        gm = jnp.maximum(jnp.max(v1), jnp.max(v2))
        gmv = _splat_f(gm)
        c1 = jnp.where(v1 == gmv, i1, _splat_i(SENT_IDX))
        c2 = jnp.where(v2 == gmv, i2, _splat_i(SENT_IDX))
        gidx = jnp.minimum(jnp.min(c1), jnp.min(c2))
        m1 = c1 == _splat_i(gidx)
        m2 = c2 == _splat_i(gidx)

        def pick(col):
            a = jnp.where(m1, plsc.load_gather(rows, [iota, _splat_i(col)]), neg)
            b = jnp.where(m2, plsc.load_gather(rows, [io2, _splat_i(col)]), neg)
            return jnp.maximum(jnp.max(a), jnp.max(b))

        return gidx, pick(0), pick(1), pick(2)

    def scan(qx, qy, qz):
        qxv = _splat_f(qx)
        qyv = _splat_f(qy)
        qzv = _splat_f(qz)

        def body(j, carry):
            maxv, maxi = carry
            for u in range(UNROLL):
                sl = pl.ds((j * UNROLL + u) * L, L)
                dx = px[sl] - qxv
                dy = py[sl] - qyv
                dz = pz[sl] - qzv
                t = dx * dx + dy * dy + dz * dz
                nd = jnp.minimum(dists[sl], t)
                dists[sl] = nd
                gi = _splat_i(base + (j * UNROLL + u) * L) + iota
                upd = nd > maxv
                maxv = jnp.where(upd, nd, maxv)
                maxi = jnp.where(upd, gi, maxi)
            return maxv, maxi

        maxv, maxi = lax.fori_loop(
            0, NV // UNROLL, body, (_splat_f(NEG_INF), _splat_i(0))
        )
        lm = jnp.max(maxv)
        lcand = jnp.where(maxv == _splat_f(lm), maxi, _splat_i(SENT_IDX))
        li = jnp.min(lcand)
        iv = _splat_i(li - base)
        cx = jnp.max(plsc.load_gather(px, [iv]))
        cy = jnp.max(plsc.load_gather(py, [iv]))
        cz = jnp.max(plsc.load_gather(pz, [iv]))
        return lm, li, cx, cy, cz

    lane0 = iota == _splat_i(0)
    sx = jnp.max(jnp.where(lane0, px[pl.ds(0, L)], neg))
    sy = jnp.max(jnp.where(lane0, py[pl.ds(0, L)], neg))
    sz = jnp.max(jnp.where(lane0, pz[pl.ds(0, L)], neg))
    is0 = w == 0
    publish(jnp.int32(0), jnp.int32(1), jnp.where(is0, POS_INF, NEG_INF),
            jnp.where(is0, 0, SENT_IDX).astype(I32), sx, sy, sz)

    def step(i, acc):
        buf = lax.rem(i, 2)
        gidx, qx, qy, qz = winner(buf, i + 1)
        pos = lax.rem(i, L)
        acc = jnp.where(iota == _splat_i(pos), _splat_i(gidx), acc)

        @pl.when(pos == L - 1)
        def _flush():
            outb[pl.ds((i // L) * L, L)] = acc

        lv, li, cx, cy, cz = scan(qx, qy, qz)
        publish(lax.rem(i + 1, 2), i + 2, lv, li, cx, cy, cz)
        return acc

    acc = lax.fori_loop(0, M - 1, step, _splat_i(0))

    gidx, _, _, _ = winner((M - 1) % 2, jnp.int32(M))
    acc = jnp.where(iota == _splat_i((M - 1) % L), _splat_i(gidx), acc)
    outb[pl.ds(((M - 1) // L) * L, L)] = acc

    publish(jnp.int32((M - 1) % 2 ^ 1), jnp.int32(M + 1),
            jnp.float32(NEG_INF), jnp.int32(SENT_IDX),
            jnp.float32(0), jnp.float32(0), jnp.float32(0))

    @pl.when(w == 0)
    def _out():
        pltpu.sync_copy(outb, out_hbm)


def kernel(feats):
    p = feats[:, :3]
    pt = jnp.pad(jnp.transpose(p), ((0, 0), (0, NPAD - N)))
    out = _fps_sc(jnp.reshape(pt, (3 * NPAD,)))
    return out[:M]


# speedup vs baseline: 1.0506x; 1.0506x over previous
import functools

import jax
import jax.numpy as jnp
from jax import lax
from jax.experimental import pallas as pl
from jax.experimental.pallas import tpu as pltpu
from jax.experimental.pallas import tpu_sc as plsc

N = 200000
M = N // 400
L = 16
NC = 2
NS = 16
NW = NC * NS
P = 6272
NV = P // L
UNROLL = 4
NPAD = NW * P
RW = 4 * L
OUT_PAD = 512

F32 = jnp.float32
I32 = jnp.int32
POS_INF = float("inf")
NEG_INF = float("-inf")
SENT_IDX = 1_000_000_000


def _splat_f(x):
    return jnp.full((L,), x, dtype=F32)


def _splat_i(x):
    return jnp.full((L,), x, dtype=I32)


_mesh = plsc.VectorSubcoreMesh(core_axis_name="c", subcore_axis_name="s")


@functools.partial(
    pl.kernel,
    out_type=jax.ShapeDtypeStruct((OUT_PAD,), I32),
    mesh=_mesh,
    scratch_types=[
        pltpu.VMEM((P,), F32),
        pltpu.VMEM((P,), F32),
        pltpu.VMEM((P,), F32),
        pltpu.VMEM((P,), F32),
        pltpu.VMEM((L,), F32),
        pltpu.VMEM((NW, RW), F32),
        pltpu.VMEM((OUT_PAD,), I32),
        pltpu.HBM((NW, RW), F32),
        pltpu.HBM((NW, RW), F32),
        pltpu.SemaphoreType.DMA,
    ],
    compiler_params=pltpu.CompilerParams(needs_layout_passes=False),
)
def _fps_sc(pt_hbm, out_hbm, px, py, pz, dists, stage, rows, outb, ex0, ex1,
            psem):
    w = lax.axis_index("c") * NS + lax.axis_index("s")
    base = w * P
    iota = lax.iota(I32, L)
    neg = _splat_f(NEG_INF)

    pltpu.sync_copy(pt_hbm.at[pl.ds(0 * NPAD + base, P)], px)
    pltpu.sync_copy(pt_hbm.at[pl.ds(1 * NPAD + base, P)], py)
    pltpu.sync_copy(pt_hbm.at[pl.ds(2 * NPAD + base, P)], pz)

    def _init(j, c):
        gi = _splat_i(base + j * L) + iota
        dists[pl.ds(j * L, L)] = jnp.where(gi < _splat_i(N), _splat_f(POS_INF),
                                           neg)
        return c

    lax.fori_loop(0, NV, _init, 0)

    def publish(buf, tag, val, idx, cx, cy, cz):
        vec = _splat_f(cx)
        vec = jnp.where(iota == _splat_i(1), _splat_f(cy), vec)
        vec = jnp.where(iota == _splat_i(2), _splat_f(cz), vec)
        vec = jnp.where(iota == _splat_i(3), _splat_f(val), vec)
        vec = jnp.where(iota == _splat_i(4), _splat_f(idx.astype(F32)), vec)
        vec = jnp.where(iota == _splat_i(5), _splat_f(tag.astype(F32)), vec)
        stage[...] = vec

        @pl.when(buf == 0)
        def _pub0():
            pltpu.async_copy(stage, ex0.at[w, pl.ds(0, L)], psem)

        @pl.when(buf == 1)
        def _pub1():
            pltpu.async_copy(stage, ex1.at[w, pl.ds(0, L)], psem)

    def drain_publish():
        pltpu.make_async_copy(stage, ex0.at[w, pl.ds(0, L)], psem).wait()

    def winner(buf, tag):
        tgt = _splat_f(tag.astype(F32))

        def _cond(ok):
            return jnp.logical_not(ok)

        def _body(ok):
            @pl.when(buf == 0)
            def _rd0():
                pltpu.sync_copy(ex0, rows)

            @pl.when(buf == 1)
            def _rd1():
                pltpu.sync_copy(ex1, rows)

            t1 = plsc.load_gather(rows, [iota, _splat_i(5)])
            t2 = plsc.load_gather(rows, [iota + _splat_i(NS), _splat_i(5)])
            return jnp.all((t1 == tgt) & (t2 == tgt))

        lax.while_loop(_cond, _body, jnp.full((), False))
        drain_publish()

        io2 = iota + _splat_i(NS)
        v1 = plsc.load_gather(rows, [iota, _splat_i(3)])
        v2 = plsc.load_gather(rows, [io2, _splat_i(3)])
        i1 = plsc.load_gather(rows, [iota, _splat_i(4)]).astype(I32)
        i2 = plsc.load_gather(rows, [io2, _splat_i(4)]).astype(I32)
        gm = jnp.maximum(jnp.max(v1), jnp.max(v2))
        gmv = _splat_f(gm)
        c1 = jnp.where(v1 == gmv, i1, _splat_i(SENT_IDX))
        c2 = jnp.where(v2 == gmv, i2, _splat_i(SENT_IDX))
        gidx = jnp.minimum(jnp.min(c1), jnp.min(c2))
        m1 = c1 == _splat_i(gidx)
        m2 = c2 == _splat_i(gidx)

        def pick(col):
            a = jnp.where(m1, plsc.load_gather(rows, [iota, _splat_i(col)]), neg)
            b = jnp.where(m2, plsc.load_gather(rows, [io2, _splat_i(col)]), neg)
            return jnp.maximum(jnp.max(a), jnp.max(b))

        return gidx, pick(0), pick(1), pick(2)

    def scan(qx, qy, qz):
        qxv = _splat_f(qx)
        qyv = _splat_f(qy)
        qzv = _splat_f(qz)

        def body(j, carry):
            maxv, maxi = carry
            for u in range(UNROLL):
                sl = pl.ds((j * UNROLL + u) * L, L)
                dx = px[sl] - qxv
                dy = py[sl] - qyv
                dz = pz[sl] - qzv
                t = dx * dx + dy * dy + dz * dz
                nd = jnp.minimum(dists[sl], t)
                dists[sl] = nd
                gi = _splat_i(base + (j * UNROLL + u) * L) + iota
                upd = nd > maxv
                maxv = jnp.where(upd, nd, maxv)
                maxi = jnp.where(upd, gi, maxi)
            return maxv, maxi

        maxv, maxi = lax.fori_loop(
            0, NV // UNROLL, body, (_splat_f(NEG_INF), _splat_i(0))
        )
        lm = jnp.max(maxv)
        lcand = jnp.where(maxv == _splat_f(lm), maxi, _splat_i(SENT_IDX))
        li = jnp.min(lcand)
        iv = _splat_i(li - base)
        cx = jnp.max(plsc.load_gather(px, [iv]))
        cy = jnp.max(plsc.load_gather(py, [iv]))
        cz = jnp.max(plsc.load_gather(pz, [iv]))
        return lm, li, cx, cy, cz

    lane0 = iota == _splat_i(0)
    sx = jnp.max(jnp.where(lane0, px[pl.ds(0, L)], neg))
    sy = jnp.max(jnp.where(lane0, py[pl.ds(0, L)], neg))
    sz = jnp.max(jnp.where(lane0, pz[pl.ds(0, L)], neg))
    is0 = w == 0
    publish(jnp.int32(0), jnp.int32(1), jnp.where(is0, POS_INF, NEG_INF),
            jnp.where(is0, 0, SENT_IDX).astype(I32), sx, sy, sz)

    def step(i, acc):
        buf = lax.rem(i, 2)
        gidx, qx, qy, qz = winner(buf, i + 1)
        pos = lax.rem(i, L)
        acc = jnp.where(iota == _splat_i(pos), _splat_i(gidx), acc)

        @pl.when(pos == L - 1)
        def _flush():
            outb[pl.ds((i // L) * L, L)] = acc

        lv, li, cx, cy, cz = scan(qx, qy, qz)
        publish(lax.rem(i + 1, 2), i + 2, lv, li, cx, cy, cz)
        return acc

    acc = lax.fori_loop(0, M - 1, step, _splat_i(0))

    gidx, _, _, _ = winner((M - 1) % 2, jnp.int32(M))
    acc = jnp.where(iota == _splat_i((M - 1) % L), _splat_i(gidx), acc)
    outb[pl.ds(((M - 1) // L) * L, L)] = acc

    publish(jnp.int32((M - 1) % 2 ^ 1), jnp.int32(M + 1),
            jnp.float32(NEG_INF), jnp.int32(SENT_IDX),
            jnp.float32(0), jnp.float32(0), jnp.float32(0))
    drain_publish()

    @pl.when(w == 0)
    def _out():
        pltpu.sync_copy(outb, out_hbm)


def kernel(feats):
    p = feats[:, :3]
    pt = jnp.pad(jnp.transpose(p), ((0, 0), (0, NPAD - N)))
    out = _fps_sc(jnp.reshape(pt, (3 * NPAD,)))
    return out[:M]


# speedup vs baseline: 1.0578x; 1.0069x over previous
import functools

import jax
import jax.numpy as jnp
from jax import lax
from jax.experimental import pallas as pl
from jax.experimental.pallas import tpu as pltpu
from jax.experimental.pallas import tpu_sc as plsc

N = 200000
M = N // 400
L = 16
NC = 2
NS = 16
NW = NC * NS
P = 6272
NV = P // L
UNROLL = 4
NPAD = NW * P
RW = 4 * L
OUT_PAD = 512

F32 = jnp.float32
I32 = jnp.int32
POS_INF = float("inf")
NEG_INF = float("-inf")
SENT_IDX = 1_000_000_000


def _splat_f(x):
    return jnp.full((L,), x, dtype=F32)


def _splat_i(x):
    return jnp.full((L,), x, dtype=I32)


_mesh = plsc.VectorSubcoreMesh(core_axis_name="c", subcore_axis_name="s")


@functools.partial(
    pl.kernel,
    out_type=jax.ShapeDtypeStruct((OUT_PAD,), I32),
    mesh=_mesh,
    scratch_types=[
        pltpu.VMEM((P,), F32),
        pltpu.VMEM((P,), F32),
        pltpu.VMEM((P,), F32),
        pltpu.VMEM((P,), F32),
        pltpu.VMEM((L,), F32),
        pltpu.VMEM((NW, RW), F32),
        pltpu.VMEM((OUT_PAD,), I32),
        pltpu.HBM((NW, RW), F32),
        pltpu.HBM((NW, RW), F32),
        pltpu.SemaphoreType.DMA,
    ],
    compiler_params=pltpu.CompilerParams(needs_layout_passes=False),
)
def _fps_sc(pt_hbm, out_hbm, px, py, pz, dists, stage, rows, outb, ex0, ex1,
            psem):
    w = lax.axis_index("c") * NS + lax.axis_index("s")
    base = w * P
    iota = lax.iota(I32, L)
    neg = _splat_f(NEG_INF)

    pltpu.sync_copy(pt_hbm.at[pl.ds(0 * NPAD + base, P)], px)
    pltpu.sync_copy(pt_hbm.at[pl.ds(1 * NPAD + base, P)], py)
    pltpu.sync_copy(pt_hbm.at[pl.ds(2 * NPAD + base, P)], pz)

    def _init(j, c):
        gi = _splat_i(base + j * L) + iota
        dists[pl.ds(j * L, L)] = jnp.where(gi < _splat_i(N), _splat_f(POS_INF),
                                           neg)
        return c

    lax.fori_loop(0, NV, _init, 0)

    def publish(buf, tag, val, idx, cx, cy, cz):
        vec = _splat_f(cx)
        vec = jnp.where(iota == _splat_i(1), _splat_f(cy), vec)
        vec = jnp.where(iota == _splat_i(2), _splat_f(cz), vec)
        vec = jnp.where(iota == _splat_i(3), _splat_f(val), vec)
        vec = jnp.where(iota == _splat_i(4), _splat_f(idx.astype(F32)), vec)
        vec = jnp.where(iota == _splat_i(5), _splat_f(tag.astype(F32)), vec)
        stage[...] = vec

        @pl.when(buf == 0)
        def _pub0():
            pltpu.async_copy(stage, ex0.at[w, pl.ds(0, L)], psem)

        @pl.when(buf == 1)
        def _pub1():
            pltpu.async_copy(stage, ex1.at[w, pl.ds(0, L)], psem)

    def drain_publish():
        pltpu.make_async_copy(stage, ex0.at[w, pl.ds(0, L)], psem).wait()

    def winner(buf, tag):
        tgt = _splat_f(tag.astype(F32))

        def _cond(ok):
            return jnp.logical_not(ok)

        def _body(ok):
            @pl.when(buf == 0)
            def _rd0():
                pltpu.sync_copy(ex0, rows)

            @pl.when(buf == 1)
            def _rd1():
                pltpu.sync_copy(ex1, rows)

            t1 = plsc.load_gather(rows, [iota, _splat_i(5)])
            t2 = plsc.load_gather(rows, [iota + _splat_i(NS), _splat_i(5)])
            return jnp.all((t1 == tgt) & (t2 == tgt))

        lax.while_loop(_cond, _body, jnp.full((), False))
        drain_publish()

        io2 = iota + _splat_i(NS)
        v1 = plsc.load_gather(rows, [iota, _splat_i(3)])
        v2 = plsc.load_gather(rows, [io2, _splat_i(3)])
        i1 = plsc.load_gather(rows, [iota, _splat_i(4)]).astype(I32)
        i2 = plsc.load_gather(rows, [io2, _splat_i(4)]).astype(I32)
        gm = jnp.max(jnp.maximum(v1, v2))
        gmv = _splat_f(gm)
        c1 = jnp.where(v1 == gmv, i1, _splat_i(SENT_IDX))
        c2 = jnp.where(v2 == gmv, i2, _splat_i(SENT_IDX))
        gidx = jnp.min(jnp.minimum(c1, c2))
        m1 = c1 == _splat_i(gidx)
        m2 = c2 == _splat_i(gidx)

        def pick(col):
            a = jnp.where(m1, plsc.load_gather(rows, [iota, _splat_i(col)]), neg)
            b = jnp.where(m2, plsc.load_gather(rows, [io2, _splat_i(col)]), neg)
            return jnp.max(jnp.maximum(a, b))

        return gidx, pick(0), pick(1), pick(2)

    def scan(qx, qy, qz):
        qxv = _splat_f(qx)
        qyv = _splat_f(qy)
        qzv = _splat_f(qz)

        def body(j, carry):
            maxv, maxi = carry
            for u in range(UNROLL):
                sl = pl.ds((j * UNROLL + u) * L, L)
                dx = px[sl] - qxv
                dy = py[sl] - qyv
                dz = pz[sl] - qzv
                t = dx * dx + dy * dy + dz * dz
                nd = jnp.minimum(dists[sl], t)
                dists[sl] = nd
                gi = _splat_i(base + (j * UNROLL + u) * L) + iota
                upd = nd > maxv
                maxv = jnp.where(upd, nd, maxv)
                maxi = jnp.where(upd, gi, maxi)
            return maxv, maxi

        maxv, maxi = lax.fori_loop(
            0, NV // UNROLL, body, (_splat_f(NEG_INF), _splat_i(0))
        )
        lm = jnp.max(maxv)
        lcand = jnp.where(maxv == _splat_f(lm), maxi, _splat_i(SENT_IDX))
        li = jnp.min(lcand)
        iv = _splat_i(li - base)
        cx = jnp.max(plsc.load_gather(px, [iv]))
        cy = jnp.max(plsc.load_gather(py, [iv]))
        cz = jnp.max(plsc.load_gather(pz, [iv]))
        return lm, li, cx, cy, cz

    lane0 = iota == _splat_i(0)
    sx = jnp.max(jnp.where(lane0, px[pl.ds(0, L)], neg))
    sy = jnp.max(jnp.where(lane0, py[pl.ds(0, L)], neg))
    sz = jnp.max(jnp.where(lane0, pz[pl.ds(0, L)], neg))
    is0 = w == 0
    publish(jnp.int32(0), jnp.int32(1), jnp.where(is0, POS_INF, NEG_INF),
            jnp.where(is0, 0, SENT_IDX).astype(I32), sx, sy, sz)

    def step(i, acc):
        buf = lax.rem(i, 2)
        gidx, qx, qy, qz = winner(buf, i + 1)
        pos = lax.rem(i, L)
        acc = jnp.where(iota == _splat_i(pos), _splat_i(gidx), acc)

        @pl.when(pos == L - 1)
        def _flush():
            outb[pl.ds((i // L) * L, L)] = acc

        lv, li, cx, cy, cz = scan(qx, qy, qz)
        publish(lax.rem(i + 1, 2), i + 2, lv, li, cx, cy, cz)
        return acc

    acc = lax.fori_loop(0, M - 1, step, _splat_i(0))

    gidx, _, _, _ = winner((M - 1) % 2, jnp.int32(M))
    acc = jnp.where(iota == _splat_i((M - 1) % L), _splat_i(gidx), acc)
    outb[pl.ds(((M - 1) // L) * L, L)] = acc

    publish(jnp.int32((M - 1) % 2 ^ 1), jnp.int32(M + 1),
            jnp.float32(NEG_INF), jnp.int32(SENT_IDX),
            jnp.float32(0), jnp.float32(0), jnp.float32(0))
    drain_publish()

    @pl.when(w == 0)
    def _out():
        pltpu.sync_copy(outb, out_hbm)


def kernel(feats):
    p = feats[:, :3]
    pt = jnp.pad(jnp.transpose(p), ((0, 0), (0, NPAD - N)))
    out = _fps_sc(jnp.reshape(pt, (3 * NPAD,)))
    return out[:M]


# speedup vs baseline: 1.0591x; 1.0012x over previous
import functools

import jax
import jax.numpy as jnp
from jax import lax
from jax.experimental import pallas as pl
from jax.experimental.pallas import tpu as pltpu
from jax.experimental.pallas import tpu_sc as plsc

N = 200000
M = N // 400
L = 16
NC = 2
NS = 16
NW = NC * NS
P = 6272
NV = P // L
UNROLL = 4
NPAD = NW * P
RW = 2 * L
OUT_PAD = 512

F32 = jnp.float32
I32 = jnp.int32
POS_INF = float("inf")
NEG_INF = float("-inf")
SENT_IDX = 1_000_000_000


def _splat_f(x):
    return jnp.full((L,), x, dtype=F32)


def _splat_i(x):
    return jnp.full((L,), x, dtype=I32)


_mesh = plsc.VectorSubcoreMesh(core_axis_name="c", subcore_axis_name="s")


@functools.partial(
    pl.kernel,
    out_type=jax.ShapeDtypeStruct((OUT_PAD,), I32),
    mesh=_mesh,
    scratch_types=[
        pltpu.VMEM((P,), F32),
        pltpu.VMEM((P,), F32),
        pltpu.VMEM((P,), F32),
        pltpu.VMEM((P,), F32),
        pltpu.VMEM((L,), F32),
        pltpu.VMEM((NW, RW), F32),
        pltpu.VMEM((OUT_PAD,), I32),
        pltpu.HBM((NW, RW), F32),
        pltpu.HBM((NW, RW), F32),
        pltpu.SemaphoreType.DMA,
    ],
    compiler_params=pltpu.CompilerParams(needs_layout_passes=False),
)
def _fps_sc(pt_hbm, out_hbm, px, py, pz, dists, stage, rows, outb, ex0, ex1,
            psem):
    w = lax.axis_index("c") * NS + lax.axis_index("s")
    base = w * P
    iota = lax.iota(I32, L)
    neg = _splat_f(NEG_INF)

    pltpu.sync_copy(pt_hbm.at[pl.ds(0 * NPAD + base, P)], px)
    pltpu.sync_copy(pt_hbm.at[pl.ds(1 * NPAD + base, P)], py)
    pltpu.sync_copy(pt_hbm.at[pl.ds(2 * NPAD + base, P)], pz)

    def _init(j, c):
        gi = _splat_i(base + j * L) + iota
        dists[pl.ds(j * L, L)] = jnp.where(gi < _splat_i(N), _splat_f(POS_INF),
                                           neg)
        return c

    lax.fori_loop(0, NV, _init, 0)

    def publish(buf, tag, val, idx, cx, cy, cz):
        vec = _splat_f(cx)
        vec = jnp.where(iota == _splat_i(1), _splat_f(cy), vec)
        vec = jnp.where(iota == _splat_i(2), _splat_f(cz), vec)
        vec = jnp.where(iota == _splat_i(3), _splat_f(val), vec)
        vec = jnp.where(iota == _splat_i(4), _splat_f(idx.astype(F32)), vec)
        vec = jnp.where(iota == _splat_i(5), _splat_f(tag.astype(F32)), vec)
        stage[...] = vec

        @pl.when(buf == 0)
        def _pub0():
            pltpu.async_copy(stage, ex0.at[w, pl.ds(0, L)], psem)

        @pl.when(buf == 1)
        def _pub1():
            pltpu.async_copy(stage, ex1.at[w, pl.ds(0, L)], psem)

    def drain_publish():
        pltpu.make_async_copy(stage, ex0.at[w, pl.ds(0, L)], psem).wait()

    def winner(buf, tag):
        tgt = _splat_f(tag.astype(F32))

        def _cond(ok):
            return jnp.logical_not(ok)

        def _body(ok):
            @pl.when(buf == 0)
            def _rd0():
                pltpu.sync_copy(ex0, rows)

            @pl.when(buf == 1)
            def _rd1():
                pltpu.sync_copy(ex1, rows)

            t1 = plsc.load_gather(rows, [iota, _splat_i(5)])
            t2 = plsc.load_gather(rows, [iota + _splat_i(NS), _splat_i(5)])
            return jnp.all((t1 == tgt) & (t2 == tgt))

        lax.while_loop(_cond, _body, jnp.full((), False))
        drain_publish()

        io2 = iota + _splat_i(NS)
        v1 = plsc.load_gather(rows, [iota, _splat_i(3)])
        v2 = plsc.load_gather(rows, [io2, _splat_i(3)])
        i1 = plsc.load_gather(rows, [iota, _splat_i(4)]).astype(I32)
        i2 = plsc.load_gather(rows, [io2, _splat_i(4)]).astype(I32)
        gm = jnp.max(jnp.maximum(v1, v2))
        gmv = _splat_f(gm)
        c1 = jnp.where(v1 == gmv, i1, _splat_i(SENT_IDX))
        c2 = jnp.where(v2 == gmv, i2, _splat_i(SENT_IDX))
        gidx = jnp.min(jnp.minimum(c1, c2))
        m1 = c1 == _splat_i(gidx)
        m2 = c2 == _splat_i(gidx)

        def pick(col):
            a = jnp.where(m1, plsc.load_gather(rows, [iota, _splat_i(col)]), neg)
            b = jnp.where(m2, plsc.load_gather(rows, [io2, _splat_i(col)]), neg)
            return jnp.max(jnp.maximum(a, b))

        return gidx, pick(0), pick(1), pick(2)

    def scan(qx, qy, qz):
        qxv = _splat_f(qx)
        qyv = _splat_f(qy)
        qzv = _splat_f(qz)

        def body(j, carry):
            maxv, maxi = carry
            for u in range(UNROLL):
                sl = pl.ds((j * UNROLL + u) * L, L)
                dx = px[sl] - qxv
                dy = py[sl] - qyv
                dz = pz[sl] - qzv
                t = dx * dx + dy * dy + dz * dz
                nd = jnp.minimum(dists[sl], t)
                dists[sl] = nd
                gi = _splat_i(base + (j * UNROLL + u) * L) + iota
                upd = nd > maxv
                maxv = jnp.where(upd, nd, maxv)
                maxi = jnp.where(upd, gi, maxi)
            return maxv, maxi

        maxv, maxi = lax.fori_loop(
            0, NV // UNROLL, body, (_splat_f(NEG_INF), _splat_i(0))
        )
        lm = jnp.max(maxv)
        lcand = jnp.where(maxv == _splat_f(lm), maxi, _splat_i(SENT_IDX))
        li = jnp.min(lcand)
        iv = _splat_i(li - base)
        cx = jnp.max(plsc.load_gather(px, [iv]))
        cy = jnp.max(plsc.load_gather(py, [iv]))
        cz = jnp.max(plsc.load_gather(pz, [iv]))
        return lm, li, cx, cy, cz

    lane0 = iota == _splat_i(0)
    sx = jnp.max(jnp.where(lane0, px[pl.ds(0, L)], neg))
    sy = jnp.max(jnp.where(lane0, py[pl.ds(0, L)], neg))
    sz = jnp.max(jnp.where(lane0, pz[pl.ds(0, L)], neg))
    is0 = w == 0
    publish(jnp.int32(0), jnp.int32(1), jnp.where(is0, POS_INF, NEG_INF),
            jnp.where(is0, 0, SENT_IDX).astype(I32), sx, sy, sz)

    def step(i, acc):
        buf = lax.rem(i, 2)
        gidx, qx, qy, qz = winner(buf, i + 1)
        pos = lax.rem(i, L)
        acc = jnp.where(iota == _splat_i(pos), _splat_i(gidx), acc)

        @pl.when(pos == L - 1)
        def _flush():
            outb[pl.ds((i // L) * L, L)] = acc

        lv, li, cx, cy, cz = scan(qx, qy, qz)
        publish(lax.rem(i + 1, 2), i + 2, lv, li, cx, cy, cz)
        return acc

    acc = lax.fori_loop(0, M - 1, step, _splat_i(0))

    gidx, _, _, _ = winner((M - 1) % 2, jnp.int32(M))
    acc = jnp.where(iota == _splat_i((M - 1) % L), _splat_i(gidx), acc)
    outb[pl.ds(((M - 1) // L) * L, L)] = acc

    publish(jnp.int32((M - 1) % 2 ^ 1), jnp.int32(M + 1),
            jnp.float32(NEG_INF), jnp.int32(SENT_IDX),
            jnp.float32(0), jnp.float32(0), jnp.float32(0))
    drain_publish()

    @pl.when(w == 0)
    def _out():
        pltpu.sync_copy(outb, out_hbm)


def kernel(feats):
    p = feats[:, :3]
    pt = jnp.pad(jnp.transpose(p), ((0, 0), (0, NPAD - N)))
    out = _fps_sc(jnp.reshape(pt, (3 * NPAD,)))
    return out[:M]
